# baseline (speedup 1.0000x reference)
import jax
import jax.numpy as jnp
from jax import lax
from jax.experimental import pallas as pl

NUM_CLASSES = 10
NUM_YAW = 2
BOX_DOF = 7
C_IN = 384
B, NY, NX = 4, 200, 176
HW = NY * NX
C_CLS = NUM_CLASSES * NUM_YAW
C_REG = C_CLS * BOX_DOF
PAD = 4
C_ALL = C_CLS + PAD + C_REG
TILE = 1408


def _mm_kernel(x_ref, w_ref, b_ref, o_ref):
    o_ref[...] = (
        lax.dot_general(
            w_ref[...], x_ref[...],
            dimension_numbers=(((0,), (1,)), ((), ())),
            preferred_element_type=jnp.float32,
        )
        + b_ref[...]
    )


def _head_matmul(xb, w_all, b_all):
    return pl.pallas_call(
        _mm_kernel,
        grid=(HW // TILE,),
        in_specs=[
            pl.BlockSpec((TILE, C_IN), lambda t: (t, 0)),
            pl.BlockSpec((C_IN, C_ALL), lambda t: (0, 0)),
            pl.BlockSpec((C_ALL, 1), lambda t: (0, 0)),
        ],
        out_specs=pl.BlockSpec((C_ALL, TILE), lambda t: (0, t)),
        out_shape=jax.ShapeDtypeStruct((C_ALL, HW), jnp.float32),
    )(xb, w_all, b_all)


def kernel(feature_map, W_cls, b_cls, W_reg, b_reg):
    xt = jnp.transpose(feature_map, (0, 2, 3, 1))
    xf = xt.reshape(B, HW, C_IN)

    perm = jnp.asarray(
        [c * 14 + d * 2 + y
         for c in range(NUM_CLASSES)
         for y in range(NUM_YAW)
         for d in range(BOX_DOF)],
        dtype=jnp.int32,
    )
    w_all = jnp.concatenate(
        [W_cls, jnp.zeros((PAD, C_IN), jnp.float32), W_reg[perm]], axis=0
    ).T
    b_all = jnp.concatenate(
        [b_cls, jnp.zeros((PAD,), jnp.float32), b_reg[perm]], axis=0
    ).reshape(C_ALL, 1)

    cls_parts = []
    reg_parts = []
    for b in range(B):
        m = _head_matmul(xf[b], w_all, b_all)
        cls_parts.append(
            m[:C_CLS].reshape(NUM_CLASSES, NUM_YAW, NY, NX)
        )
        reg_parts.append(
            jnp.transpose(
                m[C_CLS + PAD:].reshape(
                    NUM_CLASSES, NUM_YAW, BOX_DOF, NY, NX
                ),
                (0, 1, 3, 4, 2),
            )
        )
    cls_map = jnp.stack(cls_parts, axis=0)
    reg_map = jnp.stack(reg_parts, axis=0)
    return (cls_map, reg_map)


# speedup vs baseline: 2.9262x; 2.9262x over previous
import jax
import jax.numpy as jnp
from jax import lax
from jax.experimental import pallas as pl

NUM_CLASSES = 10
NUM_YAW = 2
BOX_DOF = 7
C_IN = 384
B, NY, NX = 4, 200, 176
C_CLS = NUM_CLASSES * NUM_YAW
C_REG = C_CLS * BOX_DOF
PAD = 4
C_ALL = C_CLS + PAD + C_REG
TX = 16


def _head_kernel(x_ref, w_ref, b_ref, cls_ref, reg_ref):
    x = x_ref[0].reshape(NY * TX, C_IN)
    r = lax.dot_general(
        w_ref[...], x,
        dimension_numbers=(((0,), (1,)), ((), ())),
        preferred_element_type=jnp.float32,
    ) + b_ref[...]
    r3 = r.reshape(C_ALL, NY, TX)
    v = jnp.transpose(r3, (0, 2, 1))
    cls_ref[0] = v[0:C_CLS].reshape(NUM_CLASSES, NUM_YAW, TX, NY)
    reg_ref[0] = v[C_CLS + PAD:].reshape(
        NUM_CLASSES, NUM_YAW, BOX_DOF, TX, NY
    )


def kernel(feature_map, W_cls, b_cls, W_reg, b_reg):
    xt = jnp.transpose(feature_map, (0, 2, 3, 1))

    perm = jnp.asarray(
        [c * 14 + d * 2 + y
         for c in range(NUM_CLASSES)
         for y in range(NUM_YAW)
         for d in range(BOX_DOF)],
        dtype=jnp.int32,
    )
    w_all = jnp.concatenate(
        [W_cls, jnp.zeros((PAD, C_IN), jnp.float32), W_reg[perm]], axis=0
    ).T
    b_all = jnp.concatenate(
        [b_cls, jnp.zeros((PAD,), jnp.float32), b_reg[perm]], axis=0
    ).reshape(C_ALL, 1)

    nt = NX // TX
    cls_t, reg_t = pl.pallas_call(
        _head_kernel,
        grid=(B, nt),
        in_specs=[
            pl.BlockSpec((1, NY, TX, C_IN), lambda b, t: (b, 0, t, 0)),
            pl.BlockSpec((C_IN, C_ALL), lambda b, t: (0, 0)),
            pl.BlockSpec((C_ALL, 1), lambda b, t: (0, 0)),
        ],
        out_specs=[
            pl.BlockSpec(
                (1, NUM_CLASSES, NUM_YAW, TX, NY), lambda b, t: (b, 0, 0, t, 0)
            ),
            pl.BlockSpec(
                (1, NUM_CLASSES, NUM_YAW, BOX_DOF, TX, NY),
                lambda b, t: (b, 0, 0, 0, t, 0),
            ),
        ],
        out_shape=[
            jax.ShapeDtypeStruct((B, NUM_CLASSES, NUM_YAW, NX, NY), jnp.float32),
            jax.ShapeDtypeStruct(
                (B, NUM_CLASSES, NUM_YAW, BOX_DOF, NX, NY), jnp.float32
            ),
        ],
    )(xt, w_all, b_all)

    cls_map = jnp.transpose(cls_t, (0, 1, 2, 4, 3))
    reg_map = jnp.transpose(reg_t, (0, 1, 2, 5, 4, 3))
    return (cls_map, reg_map)


# speedup vs baseline: 2.9359x; 1.0033x over previous
import jax
import jax.numpy as jnp
from jax import lax
from jax.experimental import pallas as pl

NUM_CLASSES = 10
NUM_YAW = 2
BOX_DOF = 7
C_IN = 384
B, NY, NX = 4, 200, 176
C_CLS = NUM_CLASSES * NUM_YAW
C_REG = C_CLS * BOX_DOF
PAD = 4
C_ALL = C_CLS + PAD + C_REG
TX = 16


def _head_kernel(x_ref, w_ref, b_ref, cls_ref, reg_ref):
    x = x_ref[0].reshape(NY * TX, C_IN)
    r = lax.dot_general(
        w_ref[...], x,
        dimension_numbers=(((0,), (1,)), ((), ())),
        preferred_element_type=jnp.float32,
    ) + b_ref[...]
    r3 = r.reshape(C_ALL, NY, TX)
    vc = jnp.transpose(r3[0:C_CLS], (0, 2, 1))
    cls_ref[0] = vc.reshape(NUM_CLASSES, NUM_YAW, TX, NY)
    for c in range(NUM_CLASSES):
        rows = r3[C_CLS + PAD + c * 14:C_CLS + PAD + (c + 1) * 14]
        vr = jnp.transpose(rows, (0, 2, 1))
        reg_ref[0, c] = vr.reshape(NUM_YAW, BOX_DOF, TX, NY)


def kernel(feature_map, W_cls, b_cls, W_reg, b_reg):
    xt = jnp.transpose(feature_map, (0, 2, 3, 1))

    perm = jnp.asarray(
        [c * 14 + d * 2 + y
         for c in range(NUM_CLASSES)
         for y in range(NUM_YAW)
         for d in range(BOX_DOF)],
        dtype=jnp.int32,
    )
    w_all = jnp.concatenate(
        [W_cls, jnp.zeros((PAD, C_IN), jnp.float32), W_reg[perm]], axis=0
    ).T
    b_all = jnp.concatenate(
        [b_cls, jnp.zeros((PAD,), jnp.float32), b_reg[perm]], axis=0
    ).reshape(C_ALL, 1)

    nt = NX // TX
    cls_t, reg_t = pl.pallas_call(
        _head_kernel,
        grid=(B, nt),
        in_specs=[
            pl.BlockSpec((1, NY, TX, C_IN), lambda b, t: (b, 0, t, 0)),
            pl.BlockSpec((C_IN, C_ALL), lambda b, t: (0, 0)),
            pl.BlockSpec((C_ALL, 1), lambda b, t: (0, 0)),
        ],
        out_specs=[
            pl.BlockSpec(
                (1, NUM_CLASSES, NUM_YAW, TX, NY), lambda b, t: (b, 0, 0, t, 0)
            ),
            pl.BlockSpec(
                (1, NUM_CLASSES, NUM_YAW, BOX_DOF, TX, NY),
                lambda b, t: (b, 0, 0, 0, t, 0),
            ),
        ],
        out_shape=[
            jax.ShapeDtypeStruct((B, NUM_CLASSES, NUM_YAW, NX, NY), jnp.float32),
            jax.ShapeDtypeStruct(
                (B, NUM_CLASSES, NUM_YAW, BOX_DOF, NX, NY), jnp.float32
            ),
        ],
    )(xt, w_all, b_all)

    cls_map = jnp.transpose(cls_t, (0, 1, 2, 4, 3))
    reg_map = jnp.transpose(reg_t, (0, 1, 2, 5, 4, 3))
    return (cls_map, reg_map)


# speedup vs baseline: 2.9363x; 1.0001x over previous
import jax
import jax.numpy as jnp
from jax import lax
from jax.experimental import pallas as pl
from jax.experimental.pallas import tpu as pltpu

NUM_CLASSES = 10
NUM_YAW = 2
BOX_DOF = 7
C_IN = 384
B, NY, NX = 4, 200, 176
C_CLS = NUM_CLASSES * NUM_YAW
C_REG = C_CLS * BOX_DOF
PAD = 4
C_ALL = C_CLS + PAD + C_REG
TX = 16


def _head_kernel(x_ref, w_ref, b_ref, cls_ref, reg_ref):
    x = x_ref[0].reshape(NY * TX, C_IN)

    r = lax.dot_general(
        w_ref[...], x,
        dimension_numbers=(((1,), (1,)), ((), ())),
        preferred_element_type=jnp.float32,
    ) + b_ref[...]
    r3 = r.reshape(C_ALL, NY, TX)
    vc = jnp.transpose(r3[0:C_CLS], (0, 2, 1))
    cls_ref[0] = vc.reshape(NUM_CLASSES, NUM_YAW, TX, NY)
    for c in range(NUM_CLASSES):
        rows = r3[C_CLS + PAD + c * 14:C_CLS + PAD + (c + 1) * 14]
        vr = jnp.transpose(rows, (0, 2, 1))
        reg_ref[0, c] = vr.reshape(NUM_YAW, BOX_DOF, TX, NY)


def kernel(feature_map, W_cls, b_cls, W_reg, b_reg):
    xt = jnp.transpose(feature_map, (0, 2, 3, 1))

    perm = jnp.asarray(
        [c * 14 + d * 2 + y
         for c in range(NUM_CLASSES)
         for y in range(NUM_YAW)
         for d in range(BOX_DOF)],
        dtype=jnp.int32,
    )
    w_all = jnp.concatenate(
        [W_cls, jnp.zeros((PAD, C_IN), jnp.float32), W_reg[perm]], axis=0
    )
    b_all = jnp.concatenate(
        [b_cls, jnp.zeros((PAD,), jnp.float32), b_reg[perm]], axis=0
    ).reshape(C_ALL, 1)

    nt = NX // TX
    cls_t, reg_t = pl.pallas_call(
        _head_kernel,
        grid=(B, nt),
        in_specs=[
            pl.BlockSpec((1, NY, TX, C_IN), lambda b, t: (b, 0, t, 0)),
            pl.BlockSpec((C_ALL, C_IN), lambda b, t: (0, 0)),
            pl.BlockSpec((C_ALL, 1), lambda b, t: (0, 0)),
        ],
        out_specs=[
            pl.BlockSpec(
                (1, NUM_CLASSES, NUM_YAW, TX, NY), lambda b, t: (b, 0, 0, t, 0)
            ),
            pl.BlockSpec(
                (1, NUM_CLASSES, NUM_YAW, BOX_DOF, TX, NY),
                lambda b, t: (b, 0, 0, 0, t, 0),
            ),
        ],
        out_shape=[
            jax.ShapeDtypeStruct((B, NUM_CLASSES, NUM_YAW, NX, NY), jnp.float32),
            jax.ShapeDtypeStruct(
                (B, NUM_CLASSES, NUM_YAW, BOX_DOF, NX, NY), jnp.float32
            ),
        ],
        compiler_params=pltpu.CompilerParams(
            dimension_semantics=("parallel", "parallel"),
        ),
    )(xt, w_all, b_all)

    cls_map = jnp.transpose(cls_t, (0, 1, 2, 4, 3))
    reg_map = jnp.transpose(reg_t, (0, 1, 2, 5, 4, 3))
    return (cls_map, reg_map)
